# baseline (speedup 1.0000x reference)
import functools

import jax
import jax.numpy as jnp
from jax import lax
from jax.experimental import pallas as pl
from jax.experimental.pallas import tpu as pltpu
from jax.experimental.pallas import tpu_sc as plsc

N, E, DIN, H, DOUT = 50000, 1600000, 128, 32, 64
NC, NS, L = 2, 16, 16
NW = NC * NS
NPAD = 50176
RPT = NPAD // NS
G = 128
IB = 18
OCH = 22
EW = OCH * IB * G
EPAD = NW * EW
DCH = 24
DCS = EW // DCH
ORB = 196

_mesh = plsc.VectorSubcoreMesh(core_axis_name="c", subcore_axis_name="s",
                               num_cores=NC, num_subcores=NS)
_sc_params = pltpu.CompilerParams(needs_layout_passes=False,
                                  use_tc_tiling_on_sc=False)


def _deg_body(edge_ref, out_ref, sbuf, dbuf, cnt_s, cnt_d):
    c = lax.axis_index("c")
    s = lax.axis_index("s")
    w = c * NS + s
    z16 = jnp.zeros((L,), jnp.float32)
    ones16 = jnp.ones((L,), jnp.float32)

    def zero(i, _):
        cnt_s[pl.ds(i * L, L)] = z16
        cnt_d[pl.ds(i * L, L)] = z16
        return 0

    lax.fori_loop(0, NPAD // L, zero, 0)

    def outer(k, _):
        pltpu.sync_copy(edge_ref.at[0, w * DCH + k], sbuf)
        pltpu.sync_copy(edge_ref.at[1, w * DCH + k], dbuf)

        def inner(j, _):
            sv = sbuf[pl.ds(j * L, L)]
            dv = dbuf[pl.ds(j * L, L)]
            plsc.addupdate_scatter(cnt_s, [sv], ones16)
            plsc.addupdate_scatter(cnt_d, [dv], ones16)
            return 0

        lax.fori_loop(0, DCS // L, inner, 0)
        return 0

    lax.fori_loop(0, DCH, outer, 0)
    pltpu.sync_copy(cnt_s, out_ref.at[c, s, 0])
    pltpu.sync_copy(cnt_d, out_ref.at[c, s, 1])


_deg_call = pl.kernel(
    _deg_body,
    out_type=jax.ShapeDtypeStruct((NC, NS, 2, NPAD), jnp.float32),
    mesh=_mesh,
    scratch_types=[
        pltpu.VMEM((DCS,), jnp.int32),
        pltpu.VMEM((DCS,), jnp.int32),
        pltpu.VMEM((NPAD,), jnp.float32),
        pltpu.VMEM((NPAD,), jnp.float32),
    ],
    compiler_params=_sc_params,
)


def _agg_body(edge_ref, hn_ref, out_ref, sidx, didx, rows, obuf, agg_sh, sem):
    c = lax.axis_index("c")
    s = lax.axis_index("s")
    w = c * NS + s
    z16 = jnp.zeros((L,), jnp.float32)

    def zero(i, _):
        obuf[i, pl.ds(0, L)] = z16
        obuf[i, pl.ds(L, L)] = z16
        return 0

    lax.fori_loop(0, ORB, zero, 0)
    for t in range(RPT // ORB):
        pltpu.sync_copy(obuf, agg_sh.at[pl.ds(s * RPT + t * ORB, ORB)])
    plsc.subcore_barrier()

    def chunk(k, _):
        base = w * OCH + k
        pltpu.sync_copy(edge_ref.at[0, base], sidx)
        pltpu.sync_copy(edge_ref.at[1, base], didx)

        def inner(j, _):
            pltpu.async_copy(hn_ref.at[sidx.at[j]], rows, sem).wait()
            pltpu.sync_copy(rows, agg_sh.at[didx.at[j]], add=True)
            return 0

        lax.fori_loop(0, IB, inner, 0)
        return 0

    lax.fori_loop(0, OCH, chunk, 0)
    plsc.subcore_barrier()
    for t in range(RPT // ORB):
        pltpu.sync_copy(agg_sh.at[pl.ds(s * RPT + t * ORB, ORB)], obuf)
        pltpu.sync_copy(obuf, out_ref.at[c, pl.ds(s * RPT + t * ORB, ORB)])


_agg_call = pl.kernel(
    _agg_body,
    out_type=jax.ShapeDtypeStruct((NC, NPAD, H), jnp.float32),
    mesh=_mesh,
    scratch_types=[
        pltpu.VMEM((IB, G), jnp.int32),
        pltpu.VMEM((IB, G), jnp.int32),
        pltpu.VMEM((G, H), jnp.float32),
        pltpu.VMEM((ORB, H), jnp.float32),
        pltpu.VMEM_SHARED((NPAD, H), jnp.float32),
        pltpu.SemaphoreType.DMA,
    ],
    compiler_params=_sc_params,
)

BLK = 512
GRID = NPAD // BLK


def _elu(v):
    return jnp.where(v > 0, v, jnp.exp(v) - 1.0)


def _lnorm(h, g, b):
    mu = jnp.mean(h, axis=-1, keepdims=True)
    var = jnp.mean((h - mu) ** 2, axis=-1, keepdims=True)
    return g * (h - mu) * lax.rsqrt(var + 1e-5) + b


def _inv_sqrt_deg(deg_ref):
    d = jnp.sum(deg_ref[...], axis=1, keepdims=True)
    return jnp.where(d > 0, lax.rsqrt(d), 0.0)


def _prep_body(x_ref, ds_ref, win_ref, bin_ref, gin_ref, bein_ref, out_ref):
    h = jnp.dot(x_ref[...], win_ref[...],
                preferred_element_type=jnp.float32) + bin_ref[...]
    h = _elu(_lnorm(h, gin_ref[...], bein_ref[...]))
    out_ref[...] = h * _inv_sqrt_deg(ds_ref)


def _mid_body(agg_ref, ds_ref, dd_ref, wg_ref, bg_ref, wm_ref, bm_ref, g_ref,
              be_ref, out_ref):
    a = (agg_ref[0] + agg_ref[1]) * _inv_sqrt_deg(dd_ref)
    h = _elu(jnp.dot(a, wg_ref[...],
                     preferred_element_type=jnp.float32) + bg_ref[...])
    h = jnp.dot(h, wm_ref[...], preferred_element_type=jnp.float32) + bm_ref[...]
    h = _elu(_lnorm(h, g_ref[...], be_ref[...]))
    out_ref[...] = h * _inv_sqrt_deg(ds_ref)


def _fin_body(agg_ref, dd_ref, wg_ref, bg_ref, wm_ref, bm_ref, g_ref, be_ref,
              wo_ref, bo_ref, out_ref):
    pid = pl.program_id(0)
    a = (agg_ref[0] + agg_ref[1]) * _inv_sqrt_deg(dd_ref)
    h = _elu(jnp.dot(a, wg_ref[...],
                     preferred_element_type=jnp.float32) + bg_ref[...])
    h = jnp.dot(h, wm_ref[...], preferred_element_type=jnp.float32) + bm_ref[...]
    h = _elu(_lnorm(h, g_ref[...], be_ref[...]))
    y = _elu(jnp.dot(h, wo_ref[...],
                     preferred_element_type=jnp.float32) + bo_ref[...])
    rid = pid * BLK + lax.broadcasted_iota(jnp.int32, (BLK, 1), 0)
    y = jnp.where(rid < N, y, 0.0)

    @pl.when(pid == 0)
    def _():
        out_ref[...] = jnp.zeros_like(out_ref)

    out_ref[...] += jnp.sum(y, axis=0, keepdims=True)


def _full(shape):
    return pl.BlockSpec(shape, lambda i: (0,) * len(shape))


def _rows(width):
    return pl.BlockSpec((BLK, width), lambda i: (i, 0))


_prep_call = pl.pallas_call(
    _prep_body,
    grid=(GRID,),
    in_specs=[_rows(DIN), _rows(NW), _full((DIN, H)), _full((1, H)),
              _full((1, H)), _full((1, H))],
    out_specs=_rows(H),
    out_shape=jax.ShapeDtypeStruct((NPAD, H), jnp.float32),
)

_mid_call = pl.pallas_call(
    _mid_body,
    grid=(GRID,),
    in_specs=[pl.BlockSpec((NC, BLK, H), lambda i: (0, i, 0)), _rows(NW),
              _rows(NW), _full((H, H)), _full((1, H)), _full((H, H)),
              _full((1, H)), _full((1, H)), _full((1, H))],
    out_specs=_rows(H),
    out_shape=jax.ShapeDtypeStruct((NPAD, H), jnp.float32),
)

_fin_call = pl.pallas_call(
    _fin_body,
    grid=(GRID,),
    in_specs=[pl.BlockSpec((NC, BLK, H), lambda i: (0, i, 0)), _rows(NW),
              _full((H, H)), _full((1, H)), _full((H, H)), _full((1, H)),
              _full((1, H)), _full((1, H)), _full((H, DOUT)),
              _full((1, DOUT))],
    out_specs=_full((1, DOUT)),
    out_shape=jax.ShapeDtypeStruct((1, DOUT), jnp.float32),
)


def kernel(x, edge_index, W_in, b_in, g_in, be_in, Wg1, bg1, Wm1, bm1, g1,
           be1, Wg2, bg2, Wm2, bm2, g2, be2, W_out, b_out):
    r1 = lambda v: v.reshape(1, -1)
    xp = jnp.pad(x, ((0, NPAD - N), (0, 0)))
    ep = jnp.pad(edge_index, ((0, 0), (0, EPAD - E)), constant_values=N)
    e_deg = ep.reshape(2, NW * DCH, DCS)
    e_agg = ep.reshape(2, NW * OCH, IB, G)

    deg = _deg_call(e_deg)
    ds_t = deg[:, :, 0, :].reshape(NW, NPAD).T
    dd_t = deg[:, :, 1, :].reshape(NW, NPAD).T

    hn = _prep_call(xp, ds_t, W_in, r1(b_in), r1(g_in), r1(be_in))
    agg1 = _agg_call(e_agg, hn)
    hn2 = _mid_call(agg1, ds_t, dd_t, Wg1, r1(bg1), Wm1, r1(bm1), r1(g1),
                    r1(be1))
    agg2 = _agg_call(e_agg, hn2)
    return _fin_call(agg2, dd_t, Wg2, r1(bg2), Wm2, r1(bm2), r1(g2), r1(be2),
                     W_out, r1(b_out))


# speedup vs baseline: 1.2924x; 1.2924x over previous
import functools

import jax
import jax.numpy as jnp
from jax import lax
from jax.experimental import pallas as pl
from jax.experimental.pallas import tpu as pltpu
from jax.experimental.pallas import tpu_sc as plsc

N, E, DIN, H, DOUT = 50000, 1600000, 128, 32, 64
NC, NS, L = 2, 16, 16
NW = NC * NS
NPAD = 50176
RPT = NPAD // NS
G = 128
IB = 17
OCH = 23
EW = OCH * IB * G
EPAD = NW * EW
DCH = 23
DCS = EW // DCH
ORB = 196

_mesh = plsc.VectorSubcoreMesh(core_axis_name="c", subcore_axis_name="s",
                               num_cores=NC, num_subcores=NS)
_sc_params = pltpu.CompilerParams(needs_layout_passes=False,
                                  use_tc_tiling_on_sc=False)


def _deg_body(edge_ref, out_ref, sbuf, dbuf, cnt_s, cnt_d):
    c = lax.axis_index("c")
    s = lax.axis_index("s")
    w = c * NS + s
    z16 = jnp.zeros((L,), jnp.float32)
    ones16 = jnp.ones((L,), jnp.float32)

    def zero(i, _):
        cnt_s[pl.ds(i * L, L)] = z16
        cnt_d[pl.ds(i * L, L)] = z16
        return 0

    lax.fori_loop(0, NPAD // L, zero, 0)

    def outer(k, _):
        pltpu.sync_copy(edge_ref.at[0, w * DCH + k], sbuf)
        pltpu.sync_copy(edge_ref.at[1, w * DCH + k], dbuf)

        def inner(j, _):
            sv = sbuf[pl.ds(j * L, L)]
            dv = dbuf[pl.ds(j * L, L)]
            plsc.addupdate_scatter(cnt_s, [sv], ones16)
            plsc.addupdate_scatter(cnt_d, [dv], ones16)
            return 0

        lax.fori_loop(0, DCS // L, inner, 0)
        return 0

    lax.fori_loop(0, DCH, outer, 0)
    pltpu.sync_copy(cnt_s, out_ref.at[c, s, 0])
    pltpu.sync_copy(cnt_d, out_ref.at[c, s, 1])


_deg_call = pl.kernel(
    _deg_body,
    out_type=jax.ShapeDtypeStruct((NC, NS, 2, NPAD), jnp.float32),
    mesh=_mesh,
    scratch_types=[
        pltpu.VMEM((DCS,), jnp.int32),
        pltpu.VMEM((DCS,), jnp.int32),
        pltpu.VMEM((NPAD,), jnp.float32),
        pltpu.VMEM((NPAD,), jnp.float32),
    ],
    compiler_params=_sc_params,
)


def _agg_body(edge_ref, hn_ref, out_ref, sidx, didx, rows, obuf, agg_sh, sem):
    c = lax.axis_index("c")
    s = lax.axis_index("s")
    w = c * NS + s
    z16 = jnp.zeros((L,), jnp.float32)

    def zero(i, _):
        obuf[i, pl.ds(0, L)] = z16
        obuf[i, pl.ds(L, L)] = z16
        return 0

    lax.fori_loop(0, ORB, zero, 0)
    for t in range(RPT // ORB):
        pltpu.sync_copy(obuf, agg_sh.at[pl.ds(s * RPT + t * ORB, ORB)])
    plsc.subcore_barrier()

    def chunk(k, _):
        base = w * OCH + k
        pltpu.sync_copy(edge_ref.at[0, base], sidx)
        pltpu.sync_copy(edge_ref.at[1, base], didx)

        def inner(j, _):
            pltpu.async_copy(hn_ref.at[sidx.at[j]], rows, sem).wait()
            pltpu.sync_copy(rows, agg_sh.at[didx.at[j]], add=True)
            return 0

        lax.fori_loop(0, IB, inner, 0)
        return 0

    lax.fori_loop(0, OCH, chunk, 0)
    plsc.subcore_barrier()
    for t in range(RPT // ORB):
        pltpu.sync_copy(agg_sh.at[pl.ds(s * RPT + t * ORB, ORB)], obuf)
        pltpu.sync_copy(obuf, out_ref.at[c, pl.ds(s * RPT + t * ORB, ORB)])


_agg_call = pl.kernel(
    _agg_body,
    out_type=jax.ShapeDtypeStruct((NC, NPAD, H), jnp.float32),
    mesh=_mesh,
    scratch_types=[
        pltpu.VMEM((IB, G), jnp.int32),
        pltpu.VMEM((IB, G), jnp.int32),
        pltpu.VMEM((G, H), jnp.float32),
        pltpu.VMEM((ORB, H), jnp.float32),
        pltpu.VMEM_SHARED((NPAD, H), jnp.float32),
        pltpu.SemaphoreType.DMA,
    ],
    compiler_params=_sc_params,
)

BLK = 512
GRID = NPAD // BLK


def _elu(v):
    return jnp.where(v > 0, v, jnp.exp(v) - 1.0)


def _lnorm(h, g, b):
    mu = jnp.mean(h, axis=-1, keepdims=True)
    var = jnp.mean((h - mu) ** 2, axis=-1, keepdims=True)
    return g * (h - mu) * lax.rsqrt(var + 1e-5) + b


def _inv_sqrt_deg(deg_ref):
    d = jnp.sum(deg_ref[...], axis=1, keepdims=True)
    return jnp.where(d > 0, lax.rsqrt(d), 0.0)


def _prep_body(x_ref, ds_ref, win_ref, bin_ref, gin_ref, bein_ref, out_ref):
    h = jnp.dot(x_ref[...], win_ref[...],
                preferred_element_type=jnp.float32) + bin_ref[...]
    h = _elu(_lnorm(h, gin_ref[...], bein_ref[...]))
    out_ref[...] = h * _inv_sqrt_deg(ds_ref)


def _mid_body(agg_ref, ds_ref, dd_ref, wg_ref, bg_ref, wm_ref, bm_ref, g_ref,
              be_ref, out_ref):
    a = (agg_ref[0] + agg_ref[1]) * _inv_sqrt_deg(dd_ref)
    h = _elu(jnp.dot(a, wg_ref[...],
                     preferred_element_type=jnp.float32) + bg_ref[...])
    h = jnp.dot(h, wm_ref[...], preferred_element_type=jnp.float32) + bm_ref[...]
    h = _elu(_lnorm(h, g_ref[...], be_ref[...]))
    out_ref[...] = h * _inv_sqrt_deg(ds_ref)


def _fin_body(agg_ref, dd_ref, wg_ref, bg_ref, wm_ref, bm_ref, g_ref, be_ref,
              wo_ref, bo_ref, out_ref):
    pid = pl.program_id(0)
    a = (agg_ref[0] + agg_ref[1]) * _inv_sqrt_deg(dd_ref)
    h = _elu(jnp.dot(a, wg_ref[...],
                     preferred_element_type=jnp.float32) + bg_ref[...])
    h = jnp.dot(h, wm_ref[...], preferred_element_type=jnp.float32) + bm_ref[...]
    h = _elu(_lnorm(h, g_ref[...], be_ref[...]))
    y = _elu(jnp.dot(h, wo_ref[...],
                     preferred_element_type=jnp.float32) + bo_ref[...])
    rid = pid * BLK + lax.broadcasted_iota(jnp.int32, (BLK, 1), 0)
    y = jnp.where(rid < N, y, 0.0)

    @pl.when(pid == 0)
    def _():
        out_ref[...] = jnp.zeros_like(out_ref)

    out_ref[...] += jnp.sum(y, axis=0, keepdims=True)


def _full(shape):
    return pl.BlockSpec(shape, lambda i: (0,) * len(shape))


def _rows(width):
    return pl.BlockSpec((BLK, width), lambda i: (i, 0))


_prep_call = pl.pallas_call(
    _prep_body,
    grid=(GRID,),
    in_specs=[_rows(DIN), _rows(NW), _full((DIN, H)), _full((1, H)),
              _full((1, H)), _full((1, H))],
    out_specs=_rows(H),
    out_shape=jax.ShapeDtypeStruct((NPAD, H), jnp.float32),
)

_mid_call = pl.pallas_call(
    _mid_body,
    grid=(GRID,),
    in_specs=[pl.BlockSpec((NC, BLK, H), lambda i: (0, i, 0)), _rows(NW),
              _rows(NW), _full((H, H)), _full((1, H)), _full((H, H)),
              _full((1, H)), _full((1, H)), _full((1, H))],
    out_specs=_rows(H),
    out_shape=jax.ShapeDtypeStruct((NPAD, H), jnp.float32),
)

_fin_call = pl.pallas_call(
    _fin_body,
    grid=(GRID,),
    in_specs=[pl.BlockSpec((NC, BLK, H), lambda i: (0, i, 0)), _rows(NW),
              _full((H, H)), _full((1, H)), _full((H, H)), _full((1, H)),
              _full((1, H)), _full((1, H)), _full((H, DOUT)),
              _full((1, DOUT))],
    out_specs=_full((1, DOUT)),
    out_shape=jax.ShapeDtypeStruct((1, DOUT), jnp.float32),
)


def kernel(x, edge_index, W_in, b_in, g_in, be_in, Wg1, bg1, Wm1, bm1, g1,
           be1, Wg2, bg2, Wm2, bm2, g2, be2, W_out, b_out):
    r1 = lambda v: v.reshape(1, -1)
    xp = jnp.pad(x, ((0, NPAD - N), (0, 0)))
    pad_ids = N + jnp.arange(EPAD - E, dtype=jnp.int32) % (NPAD - N)
    ep = jnp.concatenate(
        [edge_index, jnp.stack([pad_ids, pad_ids])], axis=1)
    e_deg = ep.reshape(2, NW * DCH, DCS)
    e_agg = ep.reshape(2, NW * OCH, IB, G)

    deg = _deg_call(e_deg)
    ds_t = deg[:, :, 0, :].reshape(NW, NPAD).T
    dd_t = deg[:, :, 1, :].reshape(NW, NPAD).T

    hn = _prep_call(xp, ds_t, W_in, r1(b_in), r1(g_in), r1(be_in))
    agg1 = _agg_call(e_agg, hn)
    hn2 = _mid_call(agg1, ds_t, dd_t, Wg1, r1(bg1), Wm1, r1(bm1), r1(g1),
                    r1(be1))
    agg2 = _agg_call(e_agg, hn2)
    return _fin_call(agg2, dd_t, Wg2, r1(bg2), Wm2, r1(bm2), r1(g2), r1(be2),
                     W_out, r1(b_out))


# speedup vs baseline: 1.6790x; 1.2991x over previous
import functools

import jax
import jax.numpy as jnp
from jax import lax
from jax.experimental import pallas as pl
from jax.experimental.pallas import tpu as pltpu
from jax.experimental.pallas import tpu_sc as plsc

N, E, DIN, H, DOUT = 50000, 1600000, 128, 32, 64
NC, NS, L = 2, 16, 16
NW = NC * NS
NPAD = 50176
RPT = NPAD // NS
G = 128
IB = 16
OCH = 25
EW = OCH * IB * G
EPAD = NW * EW
DCH = 25
DCS = EW // DCH
ORB = 196

_mesh = plsc.VectorSubcoreMesh(core_axis_name="c", subcore_axis_name="s",
                               num_cores=NC, num_subcores=NS)
_sc_params = pltpu.CompilerParams(needs_layout_passes=False,
                                  use_tc_tiling_on_sc=False)


def _deg_body(edge_ref, out_ref, sbuf, dbuf, cnt_s, cnt_d):
    c = lax.axis_index("c")
    s = lax.axis_index("s")
    w = c * NS + s
    z16 = jnp.zeros((L,), jnp.float32)
    ones16 = jnp.ones((L,), jnp.float32)

    def zero(i, _):
        cnt_s[pl.ds(i * L, L)] = z16
        cnt_d[pl.ds(i * L, L)] = z16
        return 0

    lax.fori_loop(0, NPAD // L, zero, 0)

    def outer(k, _):
        pltpu.sync_copy(edge_ref.at[0, w * DCH + k], sbuf)
        pltpu.sync_copy(edge_ref.at[1, w * DCH + k], dbuf)

        def inner(j, _):
            sv = sbuf[pl.ds(j * L, L)]
            dv = dbuf[pl.ds(j * L, L)]
            plsc.addupdate_scatter(cnt_s, [sv], ones16)
            plsc.addupdate_scatter(cnt_d, [dv], ones16)
            return 0

        lax.fori_loop(0, DCS // L, inner, 0)
        return 0

    lax.fori_loop(0, DCH, outer, 0)
    pltpu.sync_copy(cnt_s, out_ref.at[c, s, 0])
    pltpu.sync_copy(cnt_d, out_ref.at[c, s, 1])


_deg_call = pl.kernel(
    _deg_body,
    out_type=jax.ShapeDtypeStruct((NC, NS, 2, NPAD), jnp.float32),
    mesh=_mesh,
    scratch_types=[
        pltpu.VMEM((DCS,), jnp.int32),
        pltpu.VMEM((DCS,), jnp.int32),
        pltpu.VMEM((NPAD,), jnp.float32),
        pltpu.VMEM((NPAD,), jnp.float32),
    ],
    compiler_params=_sc_params,
)


def _agg_body(edge_ref, hn_ref, out_ref, sidx, didx, rows, obuf, agg_sh, sem):
    c = lax.axis_index("c")
    s = lax.axis_index("s")
    w = c * NS + s
    z16 = jnp.zeros((L,), jnp.float32)

    def zero(i, _):
        obuf[i, pl.ds(0, L)] = z16
        obuf[i, pl.ds(L, L)] = z16
        return 0

    lax.fori_loop(0, ORB, zero, 0)
    for t in range(RPT // ORB):
        pltpu.sync_copy(obuf, agg_sh.at[pl.ds(s * RPT + t * ORB, ORB)])
    plsc.subcore_barrier()

    def gfire(j, slot):
        return pltpu.async_copy(hn_ref.at[sidx.at[j]],
                                rows.at[pl.ds(slot * G, G)], sem)

    def gwait(j, slot):
        pltpu.make_async_copy(hn_ref.at[sidx.at[j]],
                              rows.at[pl.ds(slot * G, G)], sem).wait()

    def scat(j, slot):
        pltpu.sync_copy(rows.at[pl.ds(slot * G, G)], agg_sh.at[didx.at[j]],
                        add=True)

    def chunk(k, _):
        base = w * OCH + k
        pltpu.sync_copy(edge_ref.at[0, base], sidx)
        pltpu.sync_copy(edge_ref.at[1, base], didx)
        gfire(0, 0)
        gfire(1, 1)

        def inner(j2, _):
            j = 2 * j2
            gwait(j, 0)
            scat(j, 0)

            @pl.when(j2 < IB // 2 - 1)
            def _():
                gfire(j + 2, 0)

            gwait(j + 1, 1)
            scat(j + 1, 1)

            @pl.when(j2 < IB // 2 - 1)
            def _():
                gfire(j + 3, 1)

            return 0

        lax.fori_loop(0, IB // 2, inner, 0)
        return 0

    lax.fori_loop(0, OCH, chunk, 0)
    plsc.subcore_barrier()
    for t in range(RPT // ORB):
        pltpu.sync_copy(agg_sh.at[pl.ds(s * RPT + t * ORB, ORB)], obuf)
        pltpu.sync_copy(obuf, out_ref.at[c, pl.ds(s * RPT + t * ORB, ORB)])


_agg_call = pl.kernel(
    _agg_body,
    out_type=jax.ShapeDtypeStruct((NC, NPAD, H), jnp.float32),
    mesh=_mesh,
    scratch_types=[
        pltpu.VMEM((IB, G), jnp.int32),
        pltpu.VMEM((IB, G), jnp.int32),
        pltpu.VMEM((2 * G, H), jnp.float32),
        pltpu.VMEM((ORB, H), jnp.float32),
        pltpu.VMEM_SHARED((NPAD, H), jnp.float32),
        pltpu.SemaphoreType.DMA,
    ],
    compiler_params=_sc_params,
)

BLK = 512
GRID = NPAD // BLK


def _elu(v):
    return jnp.where(v > 0, v, jnp.exp(v) - 1.0)


def _lnorm(h, g, b):
    mu = jnp.mean(h, axis=-1, keepdims=True)
    var = jnp.mean((h - mu) ** 2, axis=-1, keepdims=True)
    return g * (h - mu) * lax.rsqrt(var + 1e-5) + b


def _inv_sqrt_deg(deg_ref):
    d = jnp.sum(deg_ref[...], axis=1, keepdims=True)
    return jnp.where(d > 0, lax.rsqrt(d), 0.0)


def _prep_body(x_ref, ds_ref, win_ref, bin_ref, gin_ref, bein_ref, out_ref):
    h = jnp.dot(x_ref[...], win_ref[...],
                preferred_element_type=jnp.float32) + bin_ref[...]
    h = _elu(_lnorm(h, gin_ref[...], bein_ref[...]))
    out_ref[...] = h * _inv_sqrt_deg(ds_ref)


def _mid_body(agg_ref, ds_ref, dd_ref, wg_ref, bg_ref, wm_ref, bm_ref, g_ref,
              be_ref, out_ref):
    a = (agg_ref[0] + agg_ref[1]) * _inv_sqrt_deg(dd_ref)
    h = _elu(jnp.dot(a, wg_ref[...],
                     preferred_element_type=jnp.float32) + bg_ref[...])
    h = jnp.dot(h, wm_ref[...], preferred_element_type=jnp.float32) + bm_ref[...]
    h = _elu(_lnorm(h, g_ref[...], be_ref[...]))
    out_ref[...] = h * _inv_sqrt_deg(ds_ref)


def _fin_body(agg_ref, dd_ref, wg_ref, bg_ref, wm_ref, bm_ref, g_ref, be_ref,
              wo_ref, bo_ref, out_ref):
    pid = pl.program_id(0)
    a = (agg_ref[0] + agg_ref[1]) * _inv_sqrt_deg(dd_ref)
    h = _elu(jnp.dot(a, wg_ref[...],
                     preferred_element_type=jnp.float32) + bg_ref[...])
    h = jnp.dot(h, wm_ref[...], preferred_element_type=jnp.float32) + bm_ref[...]
    h = _elu(_lnorm(h, g_ref[...], be_ref[...]))
    y = _elu(jnp.dot(h, wo_ref[...],
                     preferred_element_type=jnp.float32) + bo_ref[...])
    rid = pid * BLK + lax.broadcasted_iota(jnp.int32, (BLK, 1), 0)
    y = jnp.where(rid < N, y, 0.0)

    @pl.when(pid == 0)
    def _():
        out_ref[...] = jnp.zeros_like(out_ref)

    out_ref[...] += jnp.sum(y, axis=0, keepdims=True)


def _full(shape):
    return pl.BlockSpec(shape, lambda i: (0,) * len(shape))


def _rows(width):
    return pl.BlockSpec((BLK, width), lambda i: (i, 0))


_prep_call = pl.pallas_call(
    _prep_body,
    grid=(GRID,),
    in_specs=[_rows(DIN), _rows(NW), _full((DIN, H)), _full((1, H)),
              _full((1, H)), _full((1, H))],
    out_specs=_rows(H),
    out_shape=jax.ShapeDtypeStruct((NPAD, H), jnp.float32),
)

_mid_call = pl.pallas_call(
    _mid_body,
    grid=(GRID,),
    in_specs=[pl.BlockSpec((NC, BLK, H), lambda i: (0, i, 0)), _rows(NW),
              _rows(NW), _full((H, H)), _full((1, H)), _full((H, H)),
              _full((1, H)), _full((1, H)), _full((1, H))],
    out_specs=_rows(H),
    out_shape=jax.ShapeDtypeStruct((NPAD, H), jnp.float32),
)

_fin_call = pl.pallas_call(
    _fin_body,
    grid=(GRID,),
    in_specs=[pl.BlockSpec((NC, BLK, H), lambda i: (0, i, 0)), _rows(NW),
              _full((H, H)), _full((1, H)), _full((H, H)), _full((1, H)),
              _full((1, H)), _full((1, H)), _full((H, DOUT)),
              _full((1, DOUT))],
    out_specs=_full((1, DOUT)),
    out_shape=jax.ShapeDtypeStruct((1, DOUT), jnp.float32),
)


def kernel(x, edge_index, W_in, b_in, g_in, be_in, Wg1, bg1, Wm1, bm1, g1,
           be1, Wg2, bg2, Wm2, bm2, g2, be2, W_out, b_out):
    r1 = lambda v: v.reshape(1, -1)
    xp = jnp.pad(x, ((0, NPAD - N), (0, 0)))
    pad_ids = N + jnp.arange(EPAD - E, dtype=jnp.int32) % (NPAD - N)
    ep = jnp.concatenate(
        [edge_index, jnp.stack([pad_ids, pad_ids])], axis=1)
    e_deg = ep.reshape(2, NW * DCH, DCS)
    e_agg = ep.reshape(2, NW * OCH, IB, G)

    deg = _deg_call(e_deg)
    ds_t = deg[:, :, 0, :].reshape(NW, NPAD).T
    dd_t = deg[:, :, 1, :].reshape(NW, NPAD).T

    hn = _prep_call(xp, ds_t, W_in, r1(b_in), r1(g_in), r1(be_in))
    agg1 = _agg_call(e_agg, hn)
    hn2 = _mid_call(agg1, ds_t, dd_t, Wg1, r1(bg1), Wm1, r1(bm1), r1(g1),
                    r1(be1))
    agg2 = _agg_call(e_agg, hn2)
    return _fin_call(agg2, dd_t, Wg2, r1(bg2), Wm2, r1(bm2), r1(g2), r1(be2),
                     W_out, r1(b_out))


# speedup vs baseline: 1.9738x; 1.1756x over previous
import functools

import jax
import jax.numpy as jnp
from jax import lax
from jax.experimental import pallas as pl
from jax.experimental.pallas import tpu as pltpu
from jax.experimental.pallas import tpu_sc as plsc

N, E, DIN, H, DOUT = 50000, 1600000, 128, 32, 64
NC, NS, L = 2, 16, 16
NW = NC * NS
NPAD = 50176
RPT = NPAD // NS
G = 128
IB = 16
OCH = 25
EW = OCH * IB * G
EPAD = NW * EW
DCH = 25
DCS = EW // DCH
ORB = 196
NSLOT = 4

_mesh = plsc.VectorSubcoreMesh(core_axis_name="c", subcore_axis_name="s",
                               num_cores=NC, num_subcores=NS)
_sc_params = pltpu.CompilerParams(needs_layout_passes=False,
                                  use_tc_tiling_on_sc=False)


def _deg_body(edge_ref, out_ref, sbuf, dbuf, cnt_s, cnt_d):
    c = lax.axis_index("c")
    s = lax.axis_index("s")
    w = c * NS + s
    z16 = jnp.zeros((L,), jnp.float32)
    ones16 = jnp.ones((L,), jnp.float32)

    def zero(i, _):
        cnt_s[pl.ds(i * L, L)] = z16
        cnt_d[pl.ds(i * L, L)] = z16
        return 0

    lax.fori_loop(0, NPAD // L, zero, 0)

    def outer(k, _):
        pltpu.sync_copy(edge_ref.at[0, w * DCH + k], sbuf)
        pltpu.sync_copy(edge_ref.at[1, w * DCH + k], dbuf)

        def inner(j, _):
            sv = sbuf[pl.ds(j * L, L)]
            dv = dbuf[pl.ds(j * L, L)]
            plsc.addupdate_scatter(cnt_s, [sv], ones16)
            plsc.addupdate_scatter(cnt_d, [dv], ones16)
            return 0

        lax.fori_loop(0, DCS // L, inner, 0)
        return 0

    lax.fori_loop(0, DCH, outer, 0)
    pltpu.sync_copy(cnt_s, out_ref.at[c, s, 0])
    pltpu.sync_copy(cnt_d, out_ref.at[c, s, 1])


_deg_call = pl.kernel(
    _deg_body,
    out_type=jax.ShapeDtypeStruct((NC, NS, 2, NPAD), jnp.float32),
    mesh=_mesh,
    scratch_types=[
        pltpu.VMEM((DCS,), jnp.int32),
        pltpu.VMEM((DCS,), jnp.int32),
        pltpu.VMEM((NPAD,), jnp.float32),
        pltpu.VMEM((NPAD,), jnp.float32),
    ],
    compiler_params=_sc_params,
)


def _agg_body(edge_ref, hn_ref, out_ref, sidx, didx, rows, obuf, agg_sh, sem):
    c = lax.axis_index("c")
    s = lax.axis_index("s")
    w = c * NS + s
    z16 = jnp.zeros((L,), jnp.float32)

    def zero(i, _):
        obuf[i, pl.ds(0, L)] = z16
        obuf[i, pl.ds(L, L)] = z16
        return 0

    lax.fori_loop(0, ORB, zero, 0)
    for t in range(RPT // ORB):
        pltpu.sync_copy(obuf, agg_sh.at[pl.ds(s * RPT + t * ORB, ORB)])
    plsc.subcore_barrier()

    def gfire(j, slot):
        return pltpu.async_copy(hn_ref.at[sidx.at[j]],
                                rows.at[pl.ds(slot * G, G)], sem)

    def gwait(j, slot):
        pltpu.make_async_copy(hn_ref.at[sidx.at[j]],
                              rows.at[pl.ds(slot * G, G)], sem).wait()

    def scat(j, slot):
        pltpu.sync_copy(rows.at[pl.ds(slot * G, G)], agg_sh.at[didx.at[j]],
                        add=True)

    def chunk(k, _):
        base = w * OCH + k
        pltpu.sync_copy(edge_ref.at[0, base], sidx)
        pltpu.sync_copy(edge_ref.at[1, base], didx)
        for q in range(NSLOT):
            gfire(q, q)

        def inner(j2, _):
            j = NSLOT * j2
            for q in range(NSLOT):
                gwait(j + q, q)
                scat(j + q, q)

                @pl.when(j2 < IB // NSLOT - 1)
                def _():
                    gfire(j + NSLOT + q, q)

            return 0

        lax.fori_loop(0, IB // NSLOT, inner, 0)
        return 0

    lax.fori_loop(0, OCH, chunk, 0)
    plsc.subcore_barrier()
    for t in range(RPT // ORB):
        pltpu.sync_copy(agg_sh.at[pl.ds(s * RPT + t * ORB, ORB)], obuf)
        pltpu.sync_copy(obuf, out_ref.at[c, pl.ds(s * RPT + t * ORB, ORB)])


_agg_call = pl.kernel(
    _agg_body,
    out_type=jax.ShapeDtypeStruct((NC, NPAD, H), jnp.float32),
    mesh=_mesh,
    scratch_types=[
        pltpu.VMEM((IB, G), jnp.int32),
        pltpu.VMEM((IB, G), jnp.int32),
        pltpu.VMEM((NSLOT * G, H), jnp.float32),
        pltpu.VMEM((ORB, H), jnp.float32),
        pltpu.VMEM_SHARED((NPAD, H), jnp.float32),
        pltpu.SemaphoreType.DMA,
    ],
    compiler_params=_sc_params,
)

BLK = 512
GRID = NPAD // BLK


def _elu(v):
    return jnp.where(v > 0, v, jnp.exp(v) - 1.0)


def _lnorm(h, g, b):
    mu = jnp.mean(h, axis=-1, keepdims=True)
    var = jnp.mean((h - mu) ** 2, axis=-1, keepdims=True)
    return g * (h - mu) * lax.rsqrt(var + 1e-5) + b


def _inv_sqrt_deg(deg_ref):
    d = jnp.sum(deg_ref[...], axis=1, keepdims=True)
    return jnp.where(d > 0, lax.rsqrt(d), 0.0)


def _prep_body(x_ref, ds_ref, win_ref, bin_ref, gin_ref, bein_ref, out_ref):
    h = jnp.dot(x_ref[...], win_ref[...],
                preferred_element_type=jnp.float32) + bin_ref[...]
    h = _elu(_lnorm(h, gin_ref[...], bein_ref[...]))
    out_ref[...] = h * _inv_sqrt_deg(ds_ref)


def _mid_body(agg_ref, ds_ref, dd_ref, wg_ref, bg_ref, wm_ref, bm_ref, g_ref,
              be_ref, out_ref):
    a = (agg_ref[0] + agg_ref[1]) * _inv_sqrt_deg(dd_ref)
    h = _elu(jnp.dot(a, wg_ref[...],
                     preferred_element_type=jnp.float32) + bg_ref[...])
    h = jnp.dot(h, wm_ref[...], preferred_element_type=jnp.float32) + bm_ref[...]
    h = _elu(_lnorm(h, g_ref[...], be_ref[...]))
    out_ref[...] = h * _inv_sqrt_deg(ds_ref)


def _fin_body(agg_ref, dd_ref, wg_ref, bg_ref, wm_ref, bm_ref, g_ref, be_ref,
              wo_ref, bo_ref, out_ref):
    pid = pl.program_id(0)
    a = (agg_ref[0] + agg_ref[1]) * _inv_sqrt_deg(dd_ref)
    h = _elu(jnp.dot(a, wg_ref[...],
                     preferred_element_type=jnp.float32) + bg_ref[...])
    h = jnp.dot(h, wm_ref[...], preferred_element_type=jnp.float32) + bm_ref[...]
    h = _elu(_lnorm(h, g_ref[...], be_ref[...]))
    y = _elu(jnp.dot(h, wo_ref[...],
                     preferred_element_type=jnp.float32) + bo_ref[...])
    rid = pid * BLK + lax.broadcasted_iota(jnp.int32, (BLK, 1), 0)
    y = jnp.where(rid < N, y, 0.0)

    @pl.when(pid == 0)
    def _():
        out_ref[...] = jnp.zeros_like(out_ref)

    out_ref[...] += jnp.sum(y, axis=0, keepdims=True)


def _full(shape):
    return pl.BlockSpec(shape, lambda i: (0,) * len(shape))


def _rows(width):
    return pl.BlockSpec((BLK, width), lambda i: (i, 0))


_prep_call = pl.pallas_call(
    _prep_body,
    grid=(GRID,),
    in_specs=[_rows(DIN), _rows(NW), _full((DIN, H)), _full((1, H)),
              _full((1, H)), _full((1, H))],
    out_specs=_rows(H),
    out_shape=jax.ShapeDtypeStruct((NPAD, H), jnp.float32),
)

_mid_call = pl.pallas_call(
    _mid_body,
    grid=(GRID,),
    in_specs=[pl.BlockSpec((NC, BLK, H), lambda i: (0, i, 0)), _rows(NW),
              _rows(NW), _full((H, H)), _full((1, H)), _full((H, H)),
              _full((1, H)), _full((1, H)), _full((1, H))],
    out_specs=_rows(H),
    out_shape=jax.ShapeDtypeStruct((NPAD, H), jnp.float32),
)

_fin_call = pl.pallas_call(
    _fin_body,
    grid=(GRID,),
    in_specs=[pl.BlockSpec((NC, BLK, H), lambda i: (0, i, 0)), _rows(NW),
              _full((H, H)), _full((1, H)), _full((H, H)), _full((1, H)),
              _full((1, H)), _full((1, H)), _full((H, DOUT)),
              _full((1, DOUT))],
    out_specs=_full((1, DOUT)),
    out_shape=jax.ShapeDtypeStruct((1, DOUT), jnp.float32),
)


def kernel(x, edge_index, W_in, b_in, g_in, be_in, Wg1, bg1, Wm1, bm1, g1,
           be1, Wg2, bg2, Wm2, bm2, g2, be2, W_out, b_out):
    r1 = lambda v: v.reshape(1, -1)
    xp = jnp.pad(x, ((0, NPAD - N), (0, 0)))
    pad_ids = N + jnp.arange(EPAD - E, dtype=jnp.int32) % (NPAD - N)
    ep = jnp.concatenate(
        [edge_index, jnp.stack([pad_ids, pad_ids])], axis=1)
    e_deg = ep.reshape(2, NW * DCH, DCS)
    e_agg = ep.reshape(2, NW * OCH, IB, G)

    deg = _deg_call(e_deg)
    ds_t = deg[:, :, 0, :].reshape(NW, NPAD).T
    dd_t = deg[:, :, 1, :].reshape(NW, NPAD).T

    hn = _prep_call(xp, ds_t, W_in, r1(b_in), r1(g_in), r1(be_in))
    agg1 = _agg_call(e_agg, hn)
    hn2 = _mid_call(agg1, ds_t, dd_t, Wg1, r1(bg1), Wm1, r1(bm1), r1(g1),
                    r1(be1))
    agg2 = _agg_call(e_agg, hn2)
    return _fin_call(agg2, dd_t, Wg2, r1(bg2), Wm2, r1(bm2), r1(g2), r1(be2),
                     W_out, r1(b_out))
